# baseline (speedup 1.0000x reference)
import functools

import jax
import jax.numpy as jnp
from jax import lax
from jax.experimental import pallas as pl
from jax.experimental.pallas import tpu as pltpu
from jax.experimental.pallas import tpu_sc as plsc

_info = plsc.get_sparse_core_info()
_NC, _NS, _L = _info.num_cores, _info.num_subcores, _info.num_lanes
_NW = _NC * _NS
_SB = 128


def _transpose_kernel(vocab: int, dim: int, units_per_w: int,
                      table_t_hbm, tail_hbm, table2_hbm, src, dst,
                      gsems, osems):
    wid = lax.axis_index("s") * _NC + lax.axis_index("c")
    lane0 = lax.iota(jnp.int32, _L)
    zero16 = lane0 * 0
    n_full = vocab // _SB

    def block_off(k):
        off = jnp.minimum(k * _NW + wid, n_full - 1) * _SB
        return pl.multiple_of(off, _SB)

    def fire(k, b):
        off_v = block_off(k)
        pltpu.async_copy(table_t_hbm.at[:, pl.ds(off_v, _SB)], src[b],
                         gsems[b])

    fire(0, 0)
    fire(1, 1)

    def pair_body(i, _):
        for b in range(2):
            k = i * 2 + b
            off_v = block_off(k)
            dst_hbm = table2_hbm.at[
                pl.ds(pl.multiple_of(off_v // 2, _SB // 2), _SB // 2)]
            pltpu.make_async_copy(table_t_hbm.at[:, pl.ds(off_v, _SB)],
                                  src[b], gsems[b]).wait()

            @pl.when(i > 0)
            def _():
                pltpu.make_async_copy(dst[b], dst_hbm, osems[b]).wait()

            for kk in range(2 * dim // _L):
                p = kk // (dim // _L)
                dvec = lane0 + (kk % (dim // _L)) * _L

                @plsc.parallel_loop(0, _SB // 2, unroll=8)
                def rbody(r, p=p, dvec=dvec, b=b, kk=kk):
                    dcol = zero16 + (2 * r + p)
                    v = plsc.load_gather(src[b], [dvec, dcol])
                    dst[b][r, pl.ds(kk * _L, _L)] = v

            pltpu.async_copy(dst[b], dst_hbm, osems[b])

            @pl.when(k + 2 < units_per_w)
            def _():
                fire(k + 2, b)

        return 0

    lax.fori_loop(0, units_per_w // 2, pair_body, 0)

    for b in range(2):
        k = units_per_w - 2 + b
        off_v = block_off(k)
        dst_hbm = table2_hbm.at[
            pl.ds(pl.multiple_of(off_v // 2, _SB // 2), _SB // 2)]
        pltpu.make_async_copy(dst[b], dst_hbm, osems[b]).wait()

    tail = vocab - n_full * _SB

    if tail:
        @pl.when(wid == 0)
        def _():
            pltpu.sync_copy(tail_hbm, dst[0].at[pl.ds(0, tail // 2)])
            pltpu.sync_copy(dst[0].at[pl.ds(0, tail // 2)],
                            table2_hbm.at[pl.ds(vocab // 2 - tail // 2,
                                                tail // 2)])


def _emb_kernel(n_tok: int, dim: int, blocks_per_w: int,
                table2_hbm, tok_hbm, out_hbm,
                idx_all, idx2, pcol, rows, outt, gsems, osems):
    wid = lax.axis_index("s") * _NC + lax.axis_index("c")
    seq0 = pl.multiple_of(wid * (blocks_per_w * _SB), blocks_per_w * _SB)
    units_per_w = n_tok * blocks_per_w
    lane0 = lax.iota(jnp.int32, _L)

    pltpu.sync_copy(tok_hbm.at[:, pl.ds(seq0, blocks_per_w * _SB)], idx_all)

    def unit_coords(k):
        t = k // blocks_per_w
        sbl = k % blocks_per_w
        return t, sbl

    def prep_and_fire(k, b):
        t, sbl = unit_coords(k)
        for j in range(_SB // _L):
            v = idx_all[t, pl.ds(sbl * _SB + j * _L, _L)]
            idx2[b][pl.ds(j * _L, _L)] = v >> 1
            pcol[b][pl.ds(j * _L, _L)] = (v & 1) << 6
        pltpu.async_copy(table2_hbm.at[idx2[b]], rows[b], gsems[b])

    prep_and_fire(0, 0)
    prep_and_fire(1, 1)

    def pair_body(i, _):
        for b in range(2):
            k = i * 2 + b
            t, sbl = unit_coords(k)
            dst = out_hbm.at[
                t, :, pl.ds(pl.multiple_of(seq0 + sbl * _SB, _SB), _SB)]
            pltpu.make_async_copy(table2_hbm.at[idx2[b]], rows[b],
                                  gsems[b]).wait()

            @pl.when(i > 0)
            def _():
                pltpu.make_async_copy(outt[b], dst, osems[b]).wait()

            for lg in range(_SB // _L):
                lanes = lane0 + lg * _L
                pv = pcol[b][pl.ds(lg * _L, _L)]

                @plsc.parallel_loop(0, dim, unroll=8)
                def dbody(d, lanes=lanes, pv=pv, b=b, lg=lg):
                    cols = pv + d
                    v = plsc.load_gather(rows[b], [lanes, cols])
                    outt[b][d, pl.ds(lg * _L, _L)] = v * 8.0

            pltpu.async_copy(outt[b], dst, osems[b])

            @pl.when(k + 2 < units_per_w)
            def _():
                prep_and_fire(k + 2, b)

        return 0

    lax.fori_loop(0, units_per_w // 2, pair_body, 0)

    for b in range(2):
        k = units_per_w - 2 + b
        t, sbl = unit_coords(k)
        dst = out_hbm.at[t, :, pl.ds(seq0 + sbl * _SB, _SB)]
        pltpu.make_async_copy(outt[b], dst, osems[b]).wait()


def kernel(tokens, table):
    n_seq, n_tok = tokens.shape
    vocab, dim = table.shape
    blocks_per_w = n_seq // _SB // _NW

    tok_t = tokens.T.astype(jnp.int32)
    table_t = table.T
    n_full = vocab // _SB
    tail = vocab - n_full * _SB
    if tail:
        tail_src = table[n_full * _SB:].reshape(tail // 2, 2 * dim)
    else:
        tail_src = jnp.zeros((1, 2 * dim), table.dtype)

    mesh = plsc.VectorSubcoreMesh(core_axis_name="c", subcore_axis_name="s")

    units_a = -(-(vocab // _SB) // _NW)
    units_a += units_a % 2
    ka = pl.kernel(
        functools.partial(_transpose_kernel, vocab, dim, units_a),
        mesh=mesh,
        out_type=jax.ShapeDtypeStruct((vocab // 2, 2 * dim), jnp.float32),
        scratch_types=[
            [pltpu.VMEM((dim, _SB), jnp.float32) for _ in range(2)],
            [pltpu.VMEM((_SB // 2, 2 * dim), jnp.float32) for _ in range(2)],
            [pltpu.SemaphoreType.DMA for _ in range(2)],
            [pltpu.SemaphoreType.DMA for _ in range(2)],
        ],
        compiler_params=pltpu.CompilerParams(use_tc_tiling_on_sc=True,
                                             needs_layout_passes=False),
    )
    table2 = ka(table_t, tail_src)
    k = pl.kernel(
        functools.partial(_emb_kernel, n_tok, dim, blocks_per_w),
        mesh=mesh,
        out_type=jax.ShapeDtypeStruct((n_tok, dim, n_seq), jnp.float32),
        scratch_types=[
            pltpu.VMEM((n_tok, blocks_per_w * _SB), jnp.int32),
            [pltpu.VMEM((_SB,), jnp.int32) for _ in range(2)],
            [pltpu.VMEM((_SB,), jnp.int32) for _ in range(2)],
            [pltpu.VMEM((_SB, 2 * dim), jnp.float32) for _ in range(2)],
            [pltpu.VMEM((dim, _SB), jnp.float32) for _ in range(2)],
            [pltpu.SemaphoreType.DMA for _ in range(2)],
            [pltpu.SemaphoreType.DMA for _ in range(2)],
        ],
        compiler_params=pltpu.CompilerParams(use_tc_tiling_on_sc=True,
                                             needs_layout_passes=False),
    )
    out3d = k(table2, tok_t)
    return out3d.transpose(2, 0, 1)
